# baseline (speedup 1.0000x reference)
import functools

import jax
import jax.numpy as jnp
from jax.experimental import pallas as pl
from jax.experimental.pallas import tpu as pltpu

_BIG = 9999999.0


def _mine_kernel(e1_ref, e2_ref, l1_ref, l2_ref, out_ref):
    j = pl.program_id(1)

    e1 = e1_ref[...]
    e2 = e2_ref[...]

    inv1 = 1.0 / (jnp.sqrt(jnp.sum(e1 * e1, axis=1, keepdims=True)) + 1e-12)
    inv2 = 1.0 / (jnp.sqrt(jnp.sum(e2 * e2, axis=1, keepdims=True)) + 1e-12)

    raw = jax.lax.dot_general(
        e1, e2, (((1,), (1,)), ((), ())),
        preferred_element_type=jnp.float32,
    )
    dist = raw * inv1 * inv2.reshape(1, -1)

    sim = (l1_ref[...] == l2_ref[...]).astype(jnp.float32)
    p_tile = jnp.min(dist + _BIG * (1.0 - sim), axis=1)
    n_tile = jnp.max(dist - _BIG * sim, axis=1)

    @pl.when(j == 0)
    def _init():
        out_ref[0, :] = p_tile
        out_ref[1, :] = n_tile

    @pl.when(j != 0)
    def _fold():
        out_ref[0, :] = jnp.minimum(out_ref[0, :], p_tile)
        out_ref[1, :] = jnp.maximum(out_ref[1, :], n_tile)


@functools.partial(jax.jit, static_argnames=("bm", "bn"))
def _mine(emb1, emb2, label1, label2, bm=2048, bn=1024):
    m, k = emb1.shape
    n = emb2.shape[0]
    l1 = label1.reshape(m, 1)
    l2 = label2.reshape(1, n)
    grid = (m // bm, n // bn)
    return pl.pallas_call(
        _mine_kernel,
        grid=grid,
        in_specs=[
            pl.BlockSpec((bm, k), lambda i, j: (i, 0)),
            pl.BlockSpec((bn, k), lambda i, j: (j, 0)),
            pl.BlockSpec((bm, 1), lambda i, j: (i, 0)),
            pl.BlockSpec((1, bn), lambda i, j: (0, j)),
        ],
        out_specs=pl.BlockSpec((2, bm), lambda i, j: (0, i)),
        out_shape=jax.ShapeDtypeStruct((2, m), jnp.float32),
        compiler_params=pltpu.CompilerParams(
            dimension_semantics=("parallel", "arbitrary"),
        ),
    )(emb1, emb2, l1, l2)


def kernel(emb1, emb2, label1, label2):
    return _mine(emb1, emb2, label1, label2)


# speedup vs baseline: 1.1370x; 1.1370x over previous
import functools

import jax
import jax.numpy as jnp
from jax.experimental import pallas as pl
from jax.experimental.pallas import tpu as pltpu

_BIG = 9999999.0


def _mine_kernel(e1_ref, e2_ref, l1_ref, l2_ref, out_ref):
    j = pl.program_id(1)

    e1 = e1_ref[...]
    e2 = e2_ref[...]

    inv1 = 1.0 / (jnp.sqrt(jnp.sum(e1 * e1, axis=1, keepdims=True)) + 1e-12)
    inv2 = 1.0 / (jnp.sqrt(jnp.sum(e2 * e2, axis=1, keepdims=True)) + 1e-12)

    raw = jax.lax.dot_general(
        e1, e2, (((1,), (1,)), ((), ())),
        preferred_element_type=jnp.float32,
    )
    dist = raw * inv1 * inv2.reshape(1, -1)

    sim = (l1_ref[...] == l2_ref[...]).astype(jnp.float32)
    p_tile = jnp.min(dist + _BIG * (1.0 - sim), axis=1)
    n_tile = jnp.max(dist - _BIG * sim, axis=1)

    @pl.when(j == 0)
    def _init():
        out_ref[0, :] = p_tile
        out_ref[1, :] = n_tile

    @pl.when(j != 0)
    def _fold():
        out_ref[0, :] = jnp.minimum(out_ref[0, :], p_tile)
        out_ref[1, :] = jnp.maximum(out_ref[1, :], n_tile)


@functools.partial(jax.jit, static_argnames=("bm", "bn"))
def _mine(emb1, emb2, label1, label2, bm=1024, bn=2048):
    m, k = emb1.shape
    n = emb2.shape[0]
    l1 = label1.reshape(m, 1)
    l2 = label2.reshape(1, n)
    grid = (m // bm, n // bn)
    return pl.pallas_call(
        _mine_kernel,
        grid=grid,
        in_specs=[
            pl.BlockSpec((bm, k), lambda i, j: (i, 0)),
            pl.BlockSpec((bn, k), lambda i, j: (j, 0)),
            pl.BlockSpec((bm, 1), lambda i, j: (i, 0)),
            pl.BlockSpec((1, bn), lambda i, j: (0, j)),
        ],
        out_specs=pl.BlockSpec((2, bm), lambda i, j: (0, i)),
        out_shape=jax.ShapeDtypeStruct((2, m), jnp.float32),
        compiler_params=pltpu.CompilerParams(
            dimension_semantics=("parallel", "arbitrary"),
        ),
    )(emb1, emb2, l1, l2)


def kernel(emb1, emb2, label1, label2):
    return _mine(emb1, emb2, label1, label2)
